# Initial kernel scaffold; baseline (speedup 1.0000x reference)
import functools

import jax
import jax.numpy as jnp
from jax import lax
from jax.experimental import pallas as pl
from jax.experimental.pallas import tpu as pltpu
from jax.experimental.pallas import tpu_sc as plsc

NC = 2
NS = 16
NW = NC * NS


def _sc_mesh():
  return plsc.VectorSubcoreMesh(
      core_axis_name="c", subcore_axis_name="s", num_cores=NC,
      num_subcores=NS)


def _make_sc_deg(n, e, chunk):
  tile_e = e // NW
  nsteps = tile_e // chunk
  assert nsteps * chunk == tile_e
  rows_per_tile = n // NS

  @functools.partial(
      pl.kernel,
      out_type=jax.ShapeDtypeStruct((NC, n, 4), jnp.float32),
      mesh=_sc_mesh(),
      scratch_types=[
          pltpu.VMEM_SHARED((n, 4), jnp.float32),
          pltpu.VMEM((chunk,), jnp.int32),
          pltpu.VMEM((chunk, 4), jnp.float32),
      ],
  )
  def deg_kernel(dst_hbm, zeros_hbm, ones_hbm, out_hbm, acc_sh, idx_v,
                 ones_v):
    cid = lax.axis_index("c")
    sid = lax.axis_index("s")
    wid = cid * NS + sid
    r0 = sid * rows_per_tile
    pltpu.sync_copy(zeros_hbm.at[pl.ds(r0, rows_per_tile)],
                    acc_sh.at[pl.ds(r0, rows_per_tile)])
    pltpu.sync_copy(ones_hbm, ones_v)
    plsc.subcore_barrier()
    for j in range(nsteps):
      base = wid * tile_e + j * chunk
      pltpu.sync_copy(dst_hbm.at[pl.ds(base, chunk)], idx_v)
      pltpu.sync_copy(ones_v, acc_sh.at[idx_v], add=True)
    plsc.subcore_barrier()
    pltpu.sync_copy(acc_sh.at[pl.ds(r0, rows_per_tile)],
                    out_hbm.at[cid, pl.ds(r0, rows_per_tile)])

  return deg_kernel


def _make_sc_prop(n, e, chunk):
  tile_e = e // NW
  nsteps = tile_e // chunk
  assert nsteps * chunk == tile_e
  rows_per_tile = n // NS

  @functools.partial(
      pl.kernel,
      out_type=jax.ShapeDtypeStruct((NC, n, 4), jnp.float32),
      mesh=_sc_mesh(),
      scratch_types=[
          pltpu.VMEM_SHARED((n, 4), jnp.float32),
          pltpu.VMEM((chunk,), jnp.int32),
          pltpu.VMEM((chunk,), jnp.int32),
          pltpu.VMEM((chunk, 4), jnp.float32),
          pltpu.SemaphoreType.DMA,
      ],
  )
  def prop_kernel(src_hbm, dst_hbm, y_hbm, zeros_hbm, out_hbm, acc_sh,
                  src_v, dst_v, rows_v, sem):
    cid = lax.axis_index("c")
    sid = lax.axis_index("s")
    wid = cid * NS + sid
    r0 = sid * rows_per_tile
    pltpu.sync_copy(zeros_hbm.at[pl.ds(r0, rows_per_tile)],
                    acc_sh.at[pl.ds(r0, rows_per_tile)])
    plsc.subcore_barrier()
    for j in range(nsteps):
      base = wid * tile_e + j * chunk
      pltpu.sync_copy(src_hbm.at[pl.ds(base, chunk)], src_v)
      pltpu.sync_copy(dst_hbm.at[pl.ds(base, chunk)], dst_v)
      pltpu.async_copy(y_hbm.at[src_v], rows_v, sem).wait()
      pltpu.sync_copy(rows_v, acc_sh.at[dst_v], add=True)
    plsc.subcore_barrier()
    pltpu.sync_copy(acc_sh.at[pl.ds(r0, rows_per_tile)],
                    out_hbm.at[cid, pl.ds(r0, rows_per_tile)])

  return prop_kernel


def _tc_prep(degp_flat, x_flat):
  def body(degp_ref, x_ref, dinv4_ref, y1_ref):
    deg = degp_ref[0] + degp_ref[1] + 1.0
    dinv4 = lax.rsqrt(deg)
    dinv4_ref[...] = dinv4
    y1_ref[...] = dinv4 * x_ref[...]

  m = x_flat.shape[0]
  return pl.pallas_call(
      body,
      out_shape=(jax.ShapeDtypeStruct((m, 128), jnp.float32),
                 jax.ShapeDtypeStruct((m, 128), jnp.float32)),
  )(degp_flat, x_flat)


def _tc_mid(aggp, y1, dinv4, W1, b1, W2, bn):
  n = y1.shape[0]
  nblk = n // bn

  def body(aggp_ref, y1_ref, dinv4_ref, w1_ref, b1_ref, w2_ref, y2_ref):
    dinv4 = dinv4_ref[...]
    z = dinv4 * (aggp_ref[0] + aggp_ref[1] + y1_ref[...])
    w1 = w1_ref[...]
    h = (z[:, 0:1] * w1[0:1, :] + z[:, 1:2] * w1[1:2, :]
         + z[:, 2:3] * w1[2:3, :] + z[:, 3:4] * w1[3:4, :])
    h = jnp.maximum(h + b1_ref[...], 0.0)
    y2_ref[...] = dinv4 * jnp.dot(h, w2_ref[...],
                                  preferred_element_type=jnp.float32)

  return pl.pallas_call(
      body,
      grid=(nblk,),
      in_specs=[
          pl.BlockSpec((2, bn, 4), lambda i: (0, i, 0)),
          pl.BlockSpec((bn, 4), lambda i: (i, 0)),
          pl.BlockSpec((bn, 4), lambda i: (i, 0)),
          pl.BlockSpec((4, 128), lambda i: (0, 0)),
          pl.BlockSpec((1, 128), lambda i: (0, 0)),
          pl.BlockSpec((128, 4), lambda i: (0, 0)),
      ],
      out_specs=pl.BlockSpec((bn, 4), lambda i: (i, 0)),
      out_shape=jax.ShapeDtypeStruct((n, 4), jnp.float32),
  )(aggp, y1, dinv4, W1, b1, W2)


def _tc_final(aggp_flat, y2_flat, dinv4_flat, b2row):
  def body(aggp_ref, y2_ref, dinv4_ref, b2_ref, out_ref):
    out_ref[...] = (dinv4_ref[...] * (aggp_ref[0] + aggp_ref[1]
                                      + y2_ref[...]) + b2_ref[...])

  m = y2_flat.shape[0]
  return pl.pallas_call(
      body,
      out_shape=jax.ShapeDtypeStruct((m, 128), jnp.float32),
  )(aggp_flat, y2_flat, dinv4_flat, b2row)


@jax.jit
def kernel(x, edge_index, W1, b1, W2, b2):
  n, in_dim = x.shape
  e = edge_index.shape[1]
  assert in_dim == 4
  m = n // 32
  chunk = 5000
  src = edge_index[0]
  dst = edge_index[1]
  zeros4 = jnp.zeros((n, 4), jnp.float32)
  ones4 = jnp.ones((chunk, 4), jnp.float32)
  b2row = jnp.reshape(jnp.tile(b2, 32), (1, 128))

  degp = _make_sc_deg(n, e, chunk)(dst, zeros4, ones4)
  dinv4_f, y1_f = _tc_prep(degp.reshape(2, m, 128), x.reshape(m, 128))
  dinv4 = dinv4_f.reshape(n, 4)
  y1 = y1_f.reshape(n, 4)

  prop = _make_sc_prop(n, e, chunk)
  agg1p = prop(src, dst, y1, zeros4)
  y2 = _tc_mid(agg1p, y1, dinv4, W1, b1.reshape(1, 128), W2, 2000)
  agg2p = prop(src, dst, y2, zeros4)
  out_f = _tc_final(agg2p.reshape(2, m, 128), y2.reshape(m, 128),
                    dinv4_f, b2row)
  return out_f.reshape(n, 4)


# speedup vs baseline: 61.3936x; 61.3936x over previous
import functools

import jax
import jax.numpy as jnp
from jax import lax
from jax.experimental import pallas as pl
from jax.experimental.pallas import tpu as pltpu
from jax.experimental.pallas import tpu_sc as plsc

NC = 2
NS = 16
NW = NC * NS
W = 8


def _sc_mesh():
  return plsc.VectorSubcoreMesh(
      core_axis_name="c", subcore_axis_name="s", num_cores=NC,
      num_subcores=NS)


_SC_PARAMS = pltpu.CompilerParams(use_tc_tiling_on_sc=False)


def _make_sc_deg(n, e, chunk):
  tile_e = e // NW
  nsteps = tile_e // chunk
  assert nsteps * chunk == tile_e
  rpt = n // NS

  @functools.partial(
      pl.kernel,
      out_type=jax.ShapeDtypeStruct((NC, n, W), jnp.float32),
      mesh=_sc_mesh(),
      scratch_types=[
          pltpu.VMEM_SHARED((n, W), jnp.float32),
          pltpu.VMEM((chunk,), jnp.int32),
          pltpu.VMEM((chunk, W), jnp.float32),
      ],
      compiler_params=_SC_PARAMS,
  )
  def deg_kernel(dst_hbm, zeros_hbm, ones_hbm, out_hbm, acc_sh, idx_v,
                 ones_v):
    cid = lax.axis_index("c")
    sid = lax.axis_index("s")
    wid = cid * NS + sid
    r0 = sid * rpt
    pltpu.sync_copy(zeros_hbm.at[pl.ds(r0, rpt)], acc_sh.at[pl.ds(r0, rpt)])
    pltpu.sync_copy(ones_hbm, ones_v)
    plsc.subcore_barrier()

    def step(j, carry):
      base = wid * tile_e + j * chunk
      pltpu.sync_copy(dst_hbm.at[pl.ds(base, chunk)], idx_v)
      pltpu.sync_copy(ones_v, acc_sh.at[idx_v], add=True)
      return carry

    lax.fori_loop(0, nsteps, step, 0)
    plsc.subcore_barrier()
    pltpu.sync_copy(acc_sh.at[pl.ds(r0, rpt)],
                    out_hbm.at[cid, pl.ds(r0, rpt)])

  return deg_kernel


def _make_sc_prop(n, e, chunk):
  tile_e = e // NW
  nsteps = tile_e // chunk
  assert nsteps * chunk == tile_e
  rpt = n // NS

  @functools.partial(
      pl.kernel,
      out_type=jax.ShapeDtypeStruct((NC, n, W), jnp.float32),
      mesh=_sc_mesh(),
      scratch_types=[
          pltpu.VMEM_SHARED((n, W), jnp.float32),
          pltpu.VMEM((chunk,), jnp.int32),
          pltpu.VMEM((chunk,), jnp.int32),
          pltpu.VMEM((chunk, W), jnp.float32),
          pltpu.SemaphoreType.DMA,
      ],
      compiler_params=_SC_PARAMS,
  )
  def prop_kernel(src_hbm, dst_hbm, y_hbm, zeros_hbm, out_hbm, acc_sh,
                  src_v, dst_v, rows_v, sem):
    cid = lax.axis_index("c")
    sid = lax.axis_index("s")
    wid = cid * NS + sid
    r0 = sid * rpt
    pltpu.sync_copy(zeros_hbm.at[pl.ds(r0, rpt)], acc_sh.at[pl.ds(r0, rpt)])
    plsc.subcore_barrier()

    def step(j, carry):
      base = wid * tile_e + j * chunk
      pltpu.sync_copy(src_hbm.at[pl.ds(base, chunk)], src_v)
      pltpu.sync_copy(dst_hbm.at[pl.ds(base, chunk)], dst_v)
      pltpu.async_copy(y_hbm.at[src_v], rows_v, sem).wait()
      pltpu.sync_copy(rows_v, acc_sh.at[dst_v], add=True)
      return carry

    lax.fori_loop(0, nsteps, step, 0)
    plsc.subcore_barrier()
    pltpu.sync_copy(acc_sh.at[pl.ds(r0, rpt)],
                    out_hbm.at[cid, pl.ds(r0, rpt)])

  return prop_kernel


def _tc_prep(degp_flat, x8_flat):
  def body(degp_ref, x_ref, dinv8_ref, y1_ref):
    deg = degp_ref[0] + degp_ref[1] + 1.0
    dinv8 = lax.rsqrt(deg)
    dinv8_ref[...] = dinv8
    y1_ref[...] = dinv8 * x_ref[...]

  m = x8_flat.shape[0]
  return pl.pallas_call(
      body,
      out_shape=(jax.ShapeDtypeStruct((m, 128), jnp.float32),
                 jax.ShapeDtypeStruct((m, 128), jnp.float32)),
  )(degp_flat, x8_flat)


def _tc_mid(aggp, y1, dinv8, W1, b1, W2p, bn):
  n = y1.shape[0]
  nblk = n // bn

  def body(aggp_ref, y1_ref, dinv8_ref, w1_ref, b1_ref, w2_ref, y2_ref):
    dinv8 = dinv8_ref[...]
    z = dinv8 * (aggp_ref[0] + aggp_ref[1] + y1_ref[...])
    w1 = w1_ref[...]
    h = (z[:, 0:1] * w1[0:1, :] + z[:, 1:2] * w1[1:2, :]
         + z[:, 2:3] * w1[2:3, :] + z[:, 3:4] * w1[3:4, :])
    h = jnp.maximum(h + b1_ref[...], 0.0)
    y2_ref[...] = dinv8 * jnp.dot(h, w2_ref[...],
                                  preferred_element_type=jnp.float32)

  return pl.pallas_call(
      body,
      grid=(nblk,),
      in_specs=[
          pl.BlockSpec((2, bn, W), lambda i: (0, i, 0)),
          pl.BlockSpec((bn, W), lambda i: (i, 0)),
          pl.BlockSpec((bn, W), lambda i: (i, 0)),
          pl.BlockSpec((4, 128), lambda i: (0, 0)),
          pl.BlockSpec((1, 128), lambda i: (0, 0)),
          pl.BlockSpec((128, W), lambda i: (0, 0)),
      ],
      out_specs=pl.BlockSpec((bn, W), lambda i: (i, 0)),
      out_shape=jax.ShapeDtypeStruct((n, W), jnp.float32),
  )(aggp, y1, dinv8, W1, b1, W2p)


def _tc_final(aggp_flat, y2_flat, dinv8_flat, b2row):
  def body(aggp_ref, y2_ref, dinv8_ref, b2_ref, out_ref):
    out_ref[...] = (dinv8_ref[...] * (aggp_ref[0] + aggp_ref[1]
                                      + y2_ref[...]) + b2_ref[...])

  m = y2_flat.shape[0]
  return pl.pallas_call(
      body,
      out_shape=jax.ShapeDtypeStruct((m, 128), jnp.float32),
  )(aggp_flat, y2_flat, dinv8_flat, b2row)


@jax.jit
def kernel(x, edge_index, W1, b1, W2, b2):
  n, in_dim = x.shape
  e = edge_index.shape[1]
  assert in_dim == 4
  m = n * W // 128
  chunk = 5000
  src = edge_index[0]
  dst = edge_index[1]
  x8 = jnp.pad(x, ((0, 0), (0, W - in_dim)))
  W2p = jnp.pad(W2, ((0, 0), (0, W - in_dim)))
  zeros8 = jnp.zeros((n, W), jnp.float32)
  ones8 = jnp.ones((chunk, W), jnp.float32)
  b2row = jnp.reshape(jnp.tile(jnp.pad(b2, (0, W - in_dim)), 128 // W),
                      (1, 128))

  degp = _make_sc_deg(n, e, chunk)(dst, zeros8, ones8)
  dinv8_f, y1_f = _tc_prep(degp.reshape(2, m, 128), x8.reshape(m, 128))
  dinv8 = dinv8_f.reshape(n, W)
  y1 = y1_f.reshape(n, W)

  prop = _make_sc_prop(n, e, chunk)
  agg1p = prop(src, dst, y1, zeros8)
  y2 = _tc_mid(agg1p, y1, dinv8, W1, b1.reshape(1, 128), W2p, 2000)
  agg2p = prop(src, dst, y2, zeros8)
  out_f = _tc_final(agg2p.reshape(2, m, 128), y2.reshape(m, 128),
                    dinv8_f, b2row)
  return out_f.reshape(n, W)[:, :in_dim]


# speedup vs baseline: 62.2837x; 1.0145x over previous
import functools

import jax
import jax.numpy as jnp
from jax import lax
from jax.experimental import pallas as pl
from jax.experimental.pallas import tpu as pltpu
from jax.experimental.pallas import tpu_sc as plsc

NC = 2
NS = 16
NW = NC * NS
W = 8


def _sc_mesh():
  return plsc.VectorSubcoreMesh(
      core_axis_name="c", subcore_axis_name="s", num_cores=NC,
      num_subcores=NS)


_SC_PARAMS = pltpu.CompilerParams(use_tc_tiling_on_sc=False)


def _make_sc_deg(n, e, chunk):
  tile_e = e // NW
  nsteps = tile_e // chunk
  assert nsteps * chunk == tile_e
  rpt = n // NS

  @functools.partial(
      pl.kernel,
      out_type=jax.ShapeDtypeStruct((NC, n, W), jnp.float32),
      mesh=_sc_mesh(),
      scratch_types=[
          pltpu.VMEM_SHARED((n, W), jnp.float32),
          pltpu.VMEM((chunk,), jnp.int32),
          pltpu.VMEM((chunk,), jnp.int32),
          pltpu.VMEM((chunk, W), jnp.float32),
          pltpu.SemaphoreType.DMA,
          pltpu.SemaphoreType.DMA,
          pltpu.SemaphoreType.DMA,
          pltpu.SemaphoreType.DMA,
      ],
      compiler_params=_SC_PARAMS,
  )
  def deg_kernel(dst_hbm, zeros_hbm, ones_hbm, out_hbm, acc_sh, idx_v0,
                 idx_v1, ones_v, semi0, semi1, sems0, sems1):
    cid = lax.axis_index("c")
    sid = lax.axis_index("s")
    wid = cid * NS + sid
    r0 = sid * rpt
    idx_v = (idx_v0, idx_v1)
    semi = (semi0, semi1)
    sems = (sems0, sems1)
    pltpu.sync_copy(zeros_hbm.at[pl.ds(r0, rpt)], acc_sh.at[pl.ds(r0, rpt)])
    pltpu.sync_copy(ones_hbm, ones_v)
    plsc.subcore_barrier()

    def load(j):
      b = j % 2
      base = wid * tile_e + j * chunk
      return pltpu.async_copy(dst_hbm.at[pl.ds(base, chunk)], idx_v[b],
                              semi[b])

    d_idx = load(0)
    scat = [None, None]
    for j in range(nsteps):
      b = j % 2
      nb = (j + 1) % 2
      d_idx.wait()
      if j + 1 < nsteps:
        if scat[nb] is not None:
          scat[nb].wait()
          scat[nb] = None
        d_idx = load(j + 1)
      if scat[b] is not None:
        scat[b].wait()
      scat[b] = pltpu.async_copy(ones_v, acc_sh.at[idx_v[b]], sems[b],
                                 add=True)
    for s in scat:
      if s is not None:
        s.wait()
    plsc.subcore_barrier()
    pltpu.sync_copy(acc_sh.at[pl.ds(r0, rpt)],
                    out_hbm.at[cid, pl.ds(r0, rpt)])

  return deg_kernel


def _make_sc_prop(n, e, chunk):
  tile_e = e // NW
  nsteps = tile_e // chunk
  assert nsteps * chunk == tile_e
  rpt = n // NS

  @functools.partial(
      pl.kernel,
      out_type=jax.ShapeDtypeStruct((NC, n, W), jnp.float32),
      mesh=_sc_mesh(),
      scratch_types=[
          pltpu.VMEM_SHARED((n, W), jnp.float32),
          pltpu.VMEM((chunk,), jnp.int32),
          pltpu.VMEM((chunk,), jnp.int32),
          pltpu.VMEM((chunk,), jnp.int32),
          pltpu.VMEM((chunk,), jnp.int32),
          pltpu.VMEM((chunk, W), jnp.float32),
          pltpu.VMEM((chunk, W), jnp.float32),
          pltpu.SemaphoreType.DMA,
          pltpu.SemaphoreType.DMA,
          pltpu.SemaphoreType.DMA,
          pltpu.SemaphoreType.DMA,
          pltpu.SemaphoreType.DMA,
          pltpu.SemaphoreType.DMA,
      ],
      compiler_params=_SC_PARAMS,
  )
  def prop_kernel(src_hbm, dst_hbm, y_hbm, zeros_hbm, out_hbm, acc_sh,
                  src_v0, src_v1, dst_v0, dst_v1, rows_v0, rows_v1,
                  semi0, semi1, semg0, semg1, sems0, sems1):
    cid = lax.axis_index("c")
    sid = lax.axis_index("s")
    wid = cid * NS + sid
    r0 = sid * rpt
    src_v = (src_v0, src_v1)
    dst_v = (dst_v0, dst_v1)
    rows_v = (rows_v0, rows_v1)
    semi = (semi0, semi1)
    semg = (semg0, semg1)
    sems = (sems0, sems1)
    pltpu.sync_copy(zeros_hbm.at[pl.ds(r0, rpt)], acc_sh.at[pl.ds(r0, rpt)])
    plsc.subcore_barrier()

    def load(j):
      b = j % 2
      base = wid * tile_e + j * chunk
      return (pltpu.async_copy(src_hbm.at[pl.ds(base, chunk)], src_v[b],
                               semi[b]),
              pltpu.async_copy(dst_hbm.at[pl.ds(base, chunk)], dst_v[b],
                               semi[b]))

    d_idx = load(0)
    scat = [None, None]
    for j in range(nsteps):
      b = j % 2
      nb = (j + 1) % 2
      d_idx[0].wait()
      d_idx[1].wait()
      if j + 1 < nsteps:
        if scat[nb] is not None:
          scat[nb].wait()
          scat[nb] = None
        d_idx = load(j + 1)
      if scat[b] is not None:
        scat[b].wait()
        scat[b] = None
      pltpu.async_copy(y_hbm.at[src_v[b]], rows_v[b], semg[b]).wait()
      scat[b] = pltpu.async_copy(rows_v[b], acc_sh.at[dst_v[b]], sems[b],
                                 add=True)
    for s in scat:
      if s is not None:
        s.wait()
    plsc.subcore_barrier()
    pltpu.sync_copy(acc_sh.at[pl.ds(r0, rpt)],
                    out_hbm.at[cid, pl.ds(r0, rpt)])

  return prop_kernel


def _tc_prep(degp_flat, x8_flat):
  def body(degp_ref, x_ref, dinv8_ref, y1_ref):
    deg = degp_ref[0] + degp_ref[1] + 1.0
    dinv8 = lax.rsqrt(deg)
    dinv8_ref[...] = dinv8
    y1_ref[...] = dinv8 * x_ref[...]

  m = x8_flat.shape[0]
  return pl.pallas_call(
      body,
      out_shape=(jax.ShapeDtypeStruct((m, 128), jnp.float32),
                 jax.ShapeDtypeStruct((m, 128), jnp.float32)),
  )(degp_flat, x8_flat)


def _tc_mid(aggp, y1, dinv8, W1, b1, W2p, bn):
  n = y1.shape[0]
  nblk = n // bn

  def body(aggp_ref, y1_ref, dinv8_ref, w1_ref, b1_ref, w2_ref, y2_ref):
    dinv8 = dinv8_ref[...]
    z = dinv8 * (aggp_ref[0] + aggp_ref[1] + y1_ref[...])
    w1 = w1_ref[...]
    h = (z[:, 0:1] * w1[0:1, :] + z[:, 1:2] * w1[1:2, :]
         + z[:, 2:3] * w1[2:3, :] + z[:, 3:4] * w1[3:4, :])
    h = jnp.maximum(h + b1_ref[...], 0.0)
    y2_ref[...] = dinv8 * jnp.dot(h, w2_ref[...],
                                  preferred_element_type=jnp.float32)

  return pl.pallas_call(
      body,
      grid=(nblk,),
      in_specs=[
          pl.BlockSpec((2, bn, W), lambda i: (0, i, 0)),
          pl.BlockSpec((bn, W), lambda i: (i, 0)),
          pl.BlockSpec((bn, W), lambda i: (i, 0)),
          pl.BlockSpec((4, 128), lambda i: (0, 0)),
          pl.BlockSpec((1, 128), lambda i: (0, 0)),
          pl.BlockSpec((128, W), lambda i: (0, 0)),
      ],
      out_specs=pl.BlockSpec((bn, W), lambda i: (i, 0)),
      out_shape=jax.ShapeDtypeStruct((n, W), jnp.float32),
  )(aggp, y1, dinv8, W1, b1, W2p)


def _tc_final(aggp_flat, y2_flat, dinv8_flat, b2row):
  def body(aggp_ref, y2_ref, dinv8_ref, b2_ref, out_ref):
    out_ref[...] = (dinv8_ref[...] * (aggp_ref[0] + aggp_ref[1]
                                      + y2_ref[...]) + b2_ref[...])

  m = y2_flat.shape[0]
  return pl.pallas_call(
      body,
      out_shape=jax.ShapeDtypeStruct((m, 128), jnp.float32),
  )(aggp_flat, y2_flat, dinv8_flat, b2row)


@jax.jit
def kernel(x, edge_index, W1, b1, W2, b2):
  n, in_dim = x.shape
  e = edge_index.shape[1]
  assert in_dim == 4
  m = n * W // 128
  chunk = 2000
  src = edge_index[0]
  dst = edge_index[1]
  x8 = jnp.pad(x, ((0, 0), (0, W - in_dim)))
  W2p = jnp.pad(W2, ((0, 0), (0, W - in_dim)))
  zeros8 = jnp.zeros((n, W), jnp.float32)
  ones8 = jnp.ones((chunk, W), jnp.float32)
  b2row = jnp.reshape(jnp.tile(jnp.pad(b2, (0, W - in_dim)), 128 // W),
                      (1, 128))

  degp = _make_sc_deg(n, e, chunk)(dst, zeros8, ones8)
  dinv8_f, y1_f = _tc_prep(degp.reshape(2, m, 128), x8.reshape(m, 128))
  dinv8 = dinv8_f.reshape(n, W)
  y1 = y1_f.reshape(n, W)

  prop = _make_sc_prop(n, e, chunk)
  agg1p = prop(src, dst, y1, zeros8)
  y2 = _tc_mid(agg1p, y1, dinv8, W1, b1.reshape(1, 128), W2p, 2000)
  agg2p = prop(src, dst, y2, zeros8)
  out_f = _tc_final(agg2p.reshape(2, m, 128), y2.reshape(m, 128),
                    dinv8_f, b2row)
  return out_f.reshape(n, W)[:, :in_dim]
